# baseline (speedup 1.0000x reference)
import functools
import jax
import jax.numpy as jnp
from jax import lax
from jax.experimental import pallas as pl
from jax.experimental.pallas import tpu as pltpu
from jax.experimental.pallas import tpu_sc as plsc

N = 10000
NP = 10240
E = 20000
EP = 20480
T = 20
ED = 200
EDP = 256
EDW = EDP // 2
KH = T * EDP
F = 4000
FP = 4096
C = 128
NFB = FP // C
NG = 16
NC, NS = 2, 16
NW = NC * NS
VOCAB = 5000

_PAIRS = NP * T
_PPW = _PAIRS // NW
_GCH = 128
_GN = _PPW // _GCH

_EPW = EP // NS
_SPW = NP // NS
_UPW = _EPW + _SPW
_UC = 64
_UCH = _UPW // _UC
_DPW = EP // NS // 128
NPA = NP + 256

_mesh = plsc.VectorSubcoreMesh(
    core_axis_name="c", subcore_axis_name="s", num_cores=NC, num_subcores=NS)


def _wid(cid, sid):
    return sid * NC + cid


@functools.partial(
    pl.kernel,
    out_type=jax.ShapeDtypeStruct((_PAIRS, EDW), jnp.int32),
    mesh=_mesh,
    scratch_types=[
        pltpu.VMEM((_GN, _GCH), jnp.int32),
        pltpu.VMEM((_GCH, EDW), jnp.int32),
        pltpu.VMEM((_GCH, EDW), jnp.int32),
        pltpu.SemaphoreType.DMA,
        pltpu.SemaphoreType.DMA,
    ],
)
def _embed_gather(idx_hbm, table_hbm, out_hbm, idx_v, rows_a, rows_b, sa, sb):
    wid = _wid(lax.axis_index("c"), lax.axis_index("s"))
    base = wid * _PPW
    pltpu.sync_copy(idx_hbm.at[wid], idx_v)

    bufs = (rows_a, rows_b)
    sems = (sa, sb)
    descs = [None, None]
    descs[0] = pltpu.async_copy(table_hbm.at[idx_v.at[0]], rows_a, sa)
    for j in range(_GN):
        p = j % 2
        descs[p].wait()
        if j + 1 < _GN:
            q = (j + 1) % 2
            descs[q] = pltpu.async_copy(
                table_hbm.at[idx_v.at[j + 1]], bufs[q], sems[q])
        pltpu.sync_copy(bufs[p], out_hbm.at[pl.ds(base + j * _GCH, _GCH)])


@functools.partial(
    pl.kernel,
    out_type=jax.ShapeDtypeStruct((NP,), jnp.float32),
    mesh=_mesh,
    scratch_types=[
        pltpu.VMEM_SHARED((NPA,), jnp.float32),
        pltpu.VMEM((_DPW, 128), jnp.int32),
        pltpu.VMEM((_DPW, 128), jnp.float32),
        pltpu.VMEM((_SPW,), jnp.float32),
        pltpu.VMEM((_SPW,), jnp.float32),
    ],
)
def _degree(dst_hbm, w_hbm, ones_hbm, dis_hbm, deg_sp, dst_v, w_v, buf, dbuf):
    cid = lax.axis_index("c")
    sid = lax.axis_index("s")
    pltpu.sync_copy(dst_hbm.at[sid], dst_v)
    pltpu.sync_copy(w_hbm.at[sid], w_v)
    pltpu.sync_copy(ones_hbm.at[pl.ds(sid * _SPW, _SPW)], buf)
    pltpu.sync_copy(buf, deg_sp.at[pl.ds(sid * _SPW, _SPW)])
    plsc.subcore_barrier()

    def body(j, carry):
        pltpu.sync_copy(w_v.at[j], deg_sp.at[dst_v.at[j]], add=True)
        return carry

    lax.fori_loop(0, _DPW, body, 0)
    plsc.subcore_barrier()

    @pl.when(cid == 0)
    def _():
        pltpu.sync_copy(deg_sp.at[pl.ds(sid * _SPW, _SPW)], dbuf)
        pltpu.sync_copy(dbuf, dis_hbm.at[pl.ds(sid * _SPW, _SPW)])


@functools.partial(
    pl.kernel,
    out_type=jax.ShapeDtypeStruct((NP, FP), jnp.float32),
    mesh=_mesh,
    scratch_types=[
        pltpu.VMEM_SHARED((NPA, C), jnp.float32),
        pltpu.VMEM((_UCH, _UC), jnp.int32),
        pltpu.VMEM((_UCH, _UC), jnp.int32),
        pltpu.VMEM((_UC,), jnp.int32),
        pltpu.VMEM((_UC,), jnp.int32),
        pltpu.VMEM((_UC, C), jnp.float32),
        pltpu.VMEM((_UC, C), jnp.float32),
        pltpu.VMEM((16, C), jnp.float32),
        pltpu.VMEM((32, C), jnp.float32),
        pltpu.SemaphoreType.DMA,
        pltpu.SemaphoreType.DMA,
    ],
)
def _propagate(z_hbm, su_hbm, du_hbm, out_hbm,
               acc, su_v, du_v, sidx_a, sidx_b, gbuf_a, gbuf_b,
               zbuf, ebuf, sa, sb):
    cid = lax.axis_index("c")
    sid = lax.axis_index("s")
    pltpu.sync_copy(su_hbm.at[sid], su_v)
    pltpu.sync_copy(du_hbm.at[sid], du_v)

    zv = jnp.zeros((16,), jnp.float32)

    def zfill(r, carry):
        for k in range(C // 16):
            zbuf[r, pl.ds(k * 16, 16)] = zv
        return carry

    lax.fori_loop(0, 16, zfill, 0)

    def fb_body(f, carry):
        fbg = cid * (NFB // NC) + f

        def z_body(r, c2):
            pltpu.sync_copy(zbuf, acc.at[pl.ds(sid * (NPA // NS) + r * 16, 16)])
            return c2

        lax.fori_loop(0, NPA // NS // 16, z_body, 0)
        plsc.subcore_barrier()

        off = fbg * NP
        sbufs = (sidx_a, sidx_b)
        gbufs = (gbuf_a, gbuf_b)
        sems = (sa, sb)

        def start(j):
            p = j % 2
            for k in range(_UC // 16):
                sl = pl.ds(k * 16, 16)
                sbufs[p][sl] = su_v[j, sl] + off
            return pltpu.async_copy(z_hbm.at[sbufs[p]], gbufs[p], sems[p])

        descs = [None, None]
        descs[0] = start(0)
        for j in range(_UCH):
            p = j % 2
            descs[p].wait()
            if j + 1 < _UCH:
                descs[(j + 1) % 2] = start(j + 1)
            pltpu.sync_copy(gbufs[p], acc.at[du_v.at[j]], add=True)
        plsc.subcore_barrier()

        def o_body(r, c2):
            row0 = sid * _SPW + r * 32
            pltpu.sync_copy(acc.at[pl.ds(row0, 32)], ebuf)
            pltpu.sync_copy(ebuf, out_hbm.at[pl.ds(row0, 32),
                                             pl.ds(fbg * C, C)])
            return c2

        lax.fori_loop(0, _SPW // 32, o_body, 0)
        plsc.subcore_barrier()
        return carry

    lax.fori_loop(0, NFB // NC, fb_body, 0)


_BM, _BN = 512, 512


def _mm1_body(a_ref, b_ref, dis_ref, out_ref):
    z = jnp.dot(a_ref[...], b_ref[...], preferred_element_type=jnp.float32)
    z = z * lax.rsqrt(dis_ref[...]).reshape(_BM, 1)
    out_ref[...] = z.reshape(_BM, _BN // C, C).swapaxes(0, 1)


def _mm1(h, w1p, dis2d):
    return pl.pallas_call(
        _mm1_body,
        grid=(NP // _BM, FP // _BN),
        in_specs=[
            pl.BlockSpec((_BM, KH), lambda i, j: (i, 0)),
            pl.BlockSpec((KH, _BN), lambda i, j: (0, j)),
            pl.BlockSpec((1, 1, _BM), lambda i, j: (i, 0, 0)),
        ],
        out_specs=pl.BlockSpec((_BN // C, _BM, C), lambda i, j: (j, i, 0)),
        out_shape=jax.ShapeDtypeStruct((NFB, NP, C), jnp.float32),
    )(h, w1p, dis2d)


def _mm2_body(acc_ref, b1_ref, w2_ref, dis_ref, out_ref):
    dcol = lax.rsqrt(dis_ref[...]).reshape(_BM, 1)
    a = jnp.maximum(acc_ref[...] * dcol + b1_ref[...], 0.0)
    z = jnp.dot(a.astype(jnp.bfloat16), w2_ref[...],
                preferred_element_type=jnp.float32)
    z = z * dcol
    out_ref[...] = z.reshape(_BM, _BN // C, C).swapaxes(0, 1)


def _mm2(acc1, b1p2d, w2p, dis2d):
    return pl.pallas_call(
        _mm2_body,
        grid=(NP // _BM, FP // _BN),
        in_specs=[
            pl.BlockSpec((_BM, FP), lambda i, j: (i, 0)),
            pl.BlockSpec((1, FP), lambda i, j: (0, 0)),
            pl.BlockSpec((FP, _BN), lambda i, j: (0, j)),
            pl.BlockSpec((1, 1, _BM), lambda i, j: (i, 0, 0)),
        ],
        out_specs=pl.BlockSpec((_BN // C, _BM, C), lambda i, j: (j, i, 0)),
        out_shape=jax.ShapeDtypeStruct((NFB, NP, C), jnp.float32),
    )(acc1, b1p2d, w2p, dis2d)


def _pool_body(acc_ref, b2_ref, dis_ref, batch_ref, l1_ref, l1b_ref,
               l2_ref, l2b_ref, out_ref, gmax_ref):
    i = pl.program_id(0)
    dcol = lax.rsqrt(dis_ref[...]).reshape(_BM, 1)
    h = jnp.maximum(acc_ref[...] * dcol + b2_ref[...], 0.0)
    bcol = batch_ref[...].reshape(_BM, 1)

    @pl.when(i == 0)
    def _():
        gmax_ref[...] = jnp.full((NG, FP), -jnp.inf, jnp.float32)

    for g in range(NG):
        m = bcol == g
        vals = jnp.where(m, h, -jnp.inf)
        cur = gmax_ref[pl.ds(g, 1), :]
        gmax_ref[pl.ds(g, 1), :] = jnp.maximum(
            cur, jnp.max(vals, axis=0, keepdims=True))

    @pl.when(i == NP // _BM - 1)
    def _():
        gm = gmax_ref[...]
        t = jnp.dot(gm, l1_ref[...], preferred_element_type=jnp.float32)
        t = jnp.maximum(t + l1b_ref[...], 0.0)
        o = jnp.dot(t, l2_ref[...], preferred_element_type=jnp.float32)
        o = jnp.maximum(o + l2b_ref[...], 0.0)
        out_ref[...] = o


def _pool_mlp(acc2, b2p2d, dis2d, batch2d, l1wp, l1bp, l2wp, l2bp):
    return pl.pallas_call(
        _pool_body,
        grid=(NP // _BM,),
        in_specs=[
            pl.BlockSpec((_BM, FP), lambda i: (i, 0)),
            pl.BlockSpec((1, FP), lambda i: (0, 0)),
            pl.BlockSpec((1, 1, _BM), lambda i: (i, 0, 0)),
            pl.BlockSpec((1, 1, _BM), lambda i: (i, 0, 0)),
            pl.BlockSpec((FP, 1024), lambda i: (0, 0)),
            pl.BlockSpec((1, 1024), lambda i: (0, 0)),
            pl.BlockSpec((1024, 128), lambda i: (0, 0)),
            pl.BlockSpec((1, 128), lambda i: (0, 0)),
        ],
        out_specs=pl.BlockSpec((NG, 128), lambda i: (0, 0)),
        out_shape=jax.ShapeDtypeStruct((NG, 128), jnp.float32),
        scratch_shapes=[pltpu.VMEM((NG, FP), jnp.float32)],
    )(acc2, b2p2d, dis2d, batch2d, l1wp, l1bp, l2wp, l2bp)


def kernel(x, edge_index, batch, embed, W1, b1, W2, b2,
           lin1_w, lin1_b, lin2_w, lin2_b):
    f32 = jnp.float32
    i32 = jnp.int32
    bf16 = jnp.bfloat16

    xp = jnp.concatenate(
        [x.astype(i32), jnp.zeros((NP - N, T), i32)], axis=0)
    idx3 = xp.reshape(NW, _GN, _GCH)

    src = edge_index[0].astype(i32)
    dst = edge_index[1].astype(i32)
    srcp = jnp.concatenate([src, jnp.zeros((EP - E,), i32)])
    dstp = jnp.concatenate([dst, jnp.full((EP - E,), NP, i32)])
    selfids = jnp.arange(NP, dtype=i32)
    su3 = jnp.concatenate(
        [srcp.reshape(NS, _EPW), selfids.reshape(NS, _SPW)],
        axis=1).reshape(NS, _UCH, _UC)
    du3 = jnp.concatenate(
        [dstp.reshape(NS, _EPW), selfids.reshape(NS, _SPW)],
        axis=1).reshape(NS, _UCH, _UC)

    dst3 = dstp.reshape(NS, _DPW, 128)
    w3 = (jnp.arange(EP, dtype=i32) < E).astype(f32).reshape(NS, _DPW, 128)
    ones = jnp.ones((NP,), f32)

    batchp = jnp.concatenate([batch.astype(i32), jnp.full((NP - N,), NG, i32)])
    batch2d = batchp.reshape(NP // _BM, 1, _BM)

    w1p = jnp.pad(W1.reshape(T, ED, F), ((0, 0), (0, EDP - ED), (0, 0)))
    w1p = jnp.pad(w1p.reshape(KH, F), ((0, 0), (0, FP - F)))
    w1p = w1p.astype(bf16)
    w2p = jnp.pad(W2, ((0, FP - F), (0, FP - F))).astype(bf16)
    b1p2d = jnp.pad(b1, (0, FP - F)).reshape(1, FP)
    b2p2d = jnp.pad(b2, (0, FP - F)).reshape(1, FP)
    l1wp = jnp.pad(lin1_w, ((0, FP - F), (0, 1024 - 1000)))
    l1bp = jnp.pad(lin1_b, (0, 1024 - 1000)).reshape(1, 1024)
    l2wp = jnp.pad(lin2_w, ((0, 1024 - 1000), (0, 128 - 4)))
    l2bp = jnp.pad(lin2_b, (0, 128 - 4)).reshape(1, 128)

    embp = jnp.pad(embed, ((0, 0), (0, EDP - ED))).astype(bf16)
    emb_i32 = lax.bitcast_convert_type(
        embp.reshape(VOCAB, EDW, 2), i32)

    h_i32 = _embed_gather(idx3, emb_i32)
    h = lax.bitcast_convert_type(h_i32, bf16).reshape(NP, KH)
    dis = _degree(dst3, w3, ones)
    dis2d = dis.reshape(NP // _BM, 1, _BM)

    z1 = _mm1(h, w1p, dis2d).reshape(NFB * NP, C)
    acc1 = _propagate(z1, su3, du3)
    z2 = _mm2(acc1, b1p2d, w2p, dis2d).reshape(NFB * NP, C)
    acc2 = _propagate(z2, su3, du3)
    out = _pool_mlp(acc2, b2p2d, dis2d, batch2d, l1wp, l1bp, l2wp, l2bp)
    return out[:, :4]


# speedup vs baseline: 8.0923x; 8.0923x over previous
import functools
import jax
import jax.numpy as jnp
from jax import lax
from jax.experimental import pallas as pl
from jax.experimental.pallas import tpu as pltpu
from jax.experimental.pallas import tpu_sc as plsc

N = 10000
NP = 10240
E = 20000
EP = 20480
T = 20
ED = 200
EDP = 256
EDW = EDP // 2
KH = T * EDP
F = 4000
FP = 4096
C = 128
NFB = FP // C
NG = 16
NC, NS = 2, 16
NW = NC * NS
VOCAB = 5000

_PAIRS = NP * T
_PPW = _PAIRS // NW
_GCH = 128
_GN = _PPW // _GCH

_EPW = EP // NS
_SPW = NP // NS
_UPW = _EPW + _SPW
_UC = 64
_UCH = _UPW // _UC
_DPW = EP // NS // 128
NPA = NP + 256

_mesh = plsc.VectorSubcoreMesh(
    core_axis_name="c", subcore_axis_name="s", num_cores=NC, num_subcores=NS)


def _wid(cid, sid):
    return sid * NC + cid


@functools.partial(
    pl.kernel,
    out_type=jax.ShapeDtypeStruct((_PAIRS, EDP), jnp.float32),
    mesh=_mesh,
    scratch_types=[
        pltpu.VMEM((_GN, _GCH), jnp.int32),
        pltpu.VMEM((_GCH, EDP), jnp.float32),
        pltpu.VMEM((_GCH, EDP), jnp.float32),
        pltpu.SemaphoreType.DMA,
        pltpu.SemaphoreType.DMA,
    ],
)
def _embed_gather(idx_hbm, table_hbm, out_hbm, idx_v, rows_a, rows_b, sa, sb):
    wid = _wid(lax.axis_index("c"), lax.axis_index("s"))
    base = wid * _PPW
    pltpu.sync_copy(idx_hbm.at[wid], idx_v)

    bufs = (rows_a, rows_b)
    sems = (sa, sb)
    descs = [None, None]
    descs[0] = pltpu.async_copy(table_hbm.at[idx_v.at[0]], rows_a, sa)
    for j in range(_GN):
        p = j % 2
        descs[p].wait()
        if j + 1 < _GN:
            q = (j + 1) % 2
            descs[q] = pltpu.async_copy(
                table_hbm.at[idx_v.at[j + 1]], bufs[q], sems[q])
        pltpu.sync_copy(bufs[p], out_hbm.at[pl.ds(base + j * _GCH, _GCH)])


@functools.partial(
    pl.kernel,
    out_type=jax.ShapeDtypeStruct((NP,), jnp.float32),
    mesh=_mesh,
    scratch_types=[
        pltpu.VMEM_SHARED((NPA,), jnp.float32),
        pltpu.VMEM((_DPW, 128), jnp.int32),
        pltpu.VMEM((_DPW, 128), jnp.float32),
        pltpu.VMEM((_SPW,), jnp.float32),
        pltpu.VMEM((_SPW,), jnp.float32),
    ],
)
def _degree(dst_hbm, w_hbm, ones_hbm, dis_hbm, deg_sp, dst_v, w_v, buf, dbuf):
    cid = lax.axis_index("c")
    sid = lax.axis_index("s")
    pltpu.sync_copy(dst_hbm.at[sid], dst_v)
    pltpu.sync_copy(w_hbm.at[sid], w_v)
    pltpu.sync_copy(ones_hbm.at[pl.ds(sid * _SPW, _SPW)], buf)
    pltpu.sync_copy(buf, deg_sp.at[pl.ds(sid * _SPW, _SPW)])
    plsc.subcore_barrier()

    def body(j, carry):
        pltpu.sync_copy(w_v.at[j], deg_sp.at[dst_v.at[j]], add=True)
        return carry

    lax.fori_loop(0, _DPW, body, 0)
    plsc.subcore_barrier()

    @pl.when(cid == 0)
    def _():
        pltpu.sync_copy(deg_sp.at[pl.ds(sid * _SPW, _SPW)], dbuf)
        pltpu.sync_copy(dbuf, dis_hbm.at[pl.ds(sid * _SPW, _SPW)])


@functools.partial(
    pl.kernel,
    out_type=jax.ShapeDtypeStruct((NP, FP), jnp.float32),
    mesh=_mesh,
    scratch_types=[
        pltpu.VMEM_SHARED((NPA, C), jnp.float32),
        pltpu.VMEM((_UCH, _UC), jnp.int32),
        pltpu.VMEM((_UCH, _UC), jnp.int32),
        pltpu.VMEM((_UC,), jnp.int32),
        pltpu.VMEM((_UC,), jnp.int32),
        pltpu.VMEM((_UC, C), jnp.float32),
        pltpu.VMEM((_UC, C), jnp.float32),
        pltpu.VMEM((16, C), jnp.float32),
        pltpu.VMEM((32, C), jnp.float32),
        pltpu.SemaphoreType.DMA,
        pltpu.SemaphoreType.DMA,
    ],
)
def _propagate(z_hbm, su_hbm, du_hbm, out_hbm,
               acc, su_v, du_v, sidx_a, sidx_b, gbuf_a, gbuf_b,
               zbuf, ebuf, sa, sb):
    cid = lax.axis_index("c")
    sid = lax.axis_index("s")
    pltpu.sync_copy(su_hbm.at[sid], su_v)
    pltpu.sync_copy(du_hbm.at[sid], du_v)

    zv = jnp.zeros((16,), jnp.float32)

    def zfill(r, carry):
        for k in range(C // 16):
            zbuf[r, pl.ds(k * 16, 16)] = zv
        return carry

    lax.fori_loop(0, 16, zfill, 0)

    def fb_body(f, carry):
        fbg = cid * (NFB // NC) + f

        def z_body(r, c2):
            pltpu.sync_copy(zbuf, acc.at[pl.ds(sid * (NPA // NS) + r * 16, 16)])
            return c2

        lax.fori_loop(0, NPA // NS // 16, z_body, 0)
        plsc.subcore_barrier()

        off = fbg * NP
        sbufs = (sidx_a, sidx_b)
        gbufs = (gbuf_a, gbuf_b)
        sems = (sa, sb)

        def start(j):
            p = j % 2
            for k in range(_UC // 16):
                sl = pl.ds(k * 16, 16)
                sbufs[p][sl] = su_v[j, sl] + off
            return pltpu.async_copy(z_hbm.at[sbufs[p]], gbufs[p], sems[p])

        descs = [None, None]
        descs[0] = start(0)
        for j in range(_UCH):
            p = j % 2
            descs[p].wait()
            if j + 1 < _UCH:
                descs[(j + 1) % 2] = start(j + 1)
            pltpu.sync_copy(gbufs[p], acc.at[du_v.at[j]], add=True)
        plsc.subcore_barrier()

        def o_body(r, c2):
            row0 = sid * _SPW + r * 32
            pltpu.sync_copy(acc.at[pl.ds(row0, 32)], ebuf)
            pltpu.sync_copy(ebuf, out_hbm.at[pl.ds(row0, 32),
                                             pl.ds(fbg * C, C)])
            return c2

        lax.fori_loop(0, _SPW // 32, o_body, 0)
        plsc.subcore_barrier()
        return carry

    lax.fori_loop(0, NFB // NC, fb_body, 0)


_BM, _BN = 512, 512


def _mm1_body(a_ref, b_ref, dis_ref, out_ref):
    z = jnp.dot(a_ref[...].astype(jnp.bfloat16), b_ref[...],
                preferred_element_type=jnp.float32)
    z = z * lax.rsqrt(dis_ref[...]).reshape(_BM, 1)
    out_ref[...] = z.reshape(_BM, _BN // C, C).swapaxes(0, 1)


def _mm1(h, w1p, dis2d):
    return pl.pallas_call(
        _mm1_body,
        grid=(NP // _BM, FP // _BN),
        in_specs=[
            pl.BlockSpec((_BM, KH), lambda i, j: (i, 0)),
            pl.BlockSpec((KH, _BN), lambda i, j: (0, j)),
            pl.BlockSpec((1, 1, _BM), lambda i, j: (i, 0, 0)),
        ],
        out_specs=pl.BlockSpec((_BN // C, _BM, C), lambda i, j: (j, i, 0)),
        out_shape=jax.ShapeDtypeStruct((NFB, NP, C), jnp.float32),
    )(h, w1p, dis2d)


def _mm2_body(acc_ref, b1_ref, w2_ref, dis_ref, out_ref):
    dcol = lax.rsqrt(dis_ref[...]).reshape(_BM, 1)
    a = jnp.maximum(acc_ref[...] * dcol + b1_ref[...], 0.0)
    z = jnp.dot(a.astype(jnp.bfloat16), w2_ref[...],
                preferred_element_type=jnp.float32)
    z = z * dcol
    out_ref[...] = z.reshape(_BM, _BN // C, C).swapaxes(0, 1)


def _mm2(acc1, b1p2d, w2p, dis2d):
    return pl.pallas_call(
        _mm2_body,
        grid=(NP // _BM, FP // _BN),
        in_specs=[
            pl.BlockSpec((_BM, FP), lambda i, j: (i, 0)),
            pl.BlockSpec((1, FP), lambda i, j: (0, 0)),
            pl.BlockSpec((FP, _BN), lambda i, j: (0, j)),
            pl.BlockSpec((1, 1, _BM), lambda i, j: (i, 0, 0)),
        ],
        out_specs=pl.BlockSpec((_BN // C, _BM, C), lambda i, j: (j, i, 0)),
        out_shape=jax.ShapeDtypeStruct((NFB, NP, C), jnp.float32),
    )(acc1, b1p2d, w2p, dis2d)


def _pool_body(acc_ref, b2_ref, dis_ref, batch_ref, l1_ref, l1b_ref,
               l2_ref, l2b_ref, out_ref, gmax_ref):
    i = pl.program_id(0)
    dcol = lax.rsqrt(dis_ref[...]).reshape(_BM, 1)
    h = jnp.maximum(acc_ref[...] * dcol + b2_ref[...], 0.0)
    bcol = batch_ref[...].reshape(_BM, 1)

    @pl.when(i == 0)
    def _():
        gmax_ref[...] = jnp.full((NG, FP), -jnp.inf, jnp.float32)

    for g in range(NG):
        m = bcol == g
        vals = jnp.where(m, h, -jnp.inf)
        cur = gmax_ref[pl.ds(g, 1), :]
        gmax_ref[pl.ds(g, 1), :] = jnp.maximum(
            cur, jnp.max(vals, axis=0, keepdims=True))

    @pl.when(i == NP // _BM - 1)
    def _():
        gm = gmax_ref[...]
        t = jnp.dot(gm, l1_ref[...], preferred_element_type=jnp.float32)
        t = jnp.maximum(t + l1b_ref[...], 0.0)
        o = jnp.dot(t, l2_ref[...], preferred_element_type=jnp.float32)
        o = jnp.maximum(o + l2b_ref[...], 0.0)
        out_ref[...] = o


def _pool_mlp(acc2, b2p2d, dis2d, batch2d, l1wp, l1bp, l2wp, l2bp):
    return pl.pallas_call(
        _pool_body,
        grid=(NP // _BM,),
        in_specs=[
            pl.BlockSpec((_BM, FP), lambda i: (i, 0)),
            pl.BlockSpec((1, FP), lambda i: (0, 0)),
            pl.BlockSpec((1, 1, _BM), lambda i: (i, 0, 0)),
            pl.BlockSpec((1, 1, _BM), lambda i: (i, 0, 0)),
            pl.BlockSpec((FP, 1024), lambda i: (0, 0)),
            pl.BlockSpec((1, 1024), lambda i: (0, 0)),
            pl.BlockSpec((1024, 128), lambda i: (0, 0)),
            pl.BlockSpec((1, 128), lambda i: (0, 0)),
        ],
        out_specs=pl.BlockSpec((NG, 128), lambda i: (0, 0)),
        out_shape=jax.ShapeDtypeStruct((NG, 128), jnp.float32),
        scratch_shapes=[pltpu.VMEM((NG, FP), jnp.float32)],
    )(acc2, b2p2d, dis2d, batch2d, l1wp, l1bp, l2wp, l2bp)


def kernel(x, edge_index, batch, embed, W1, b1, W2, b2,
           lin1_w, lin1_b, lin2_w, lin2_b):
    f32 = jnp.float32
    i32 = jnp.int32
    bf16 = jnp.bfloat16

    xp = jnp.concatenate(
        [x.astype(i32), jnp.zeros((NP - N, T), i32)], axis=0)
    idx3 = xp.reshape(NW, _GN, _GCH)

    src = edge_index[0].astype(i32)
    dst = edge_index[1].astype(i32)
    srcp = jnp.concatenate([src, jnp.zeros((EP - E,), i32)])
    dstp = jnp.concatenate([dst, jnp.full((EP - E,), NP, i32)])
    selfids = jnp.arange(NP, dtype=i32)
    su3 = jnp.concatenate(
        [srcp.reshape(NS, _EPW), selfids.reshape(NS, _SPW)],
        axis=1).reshape(NS, _UCH, _UC)
    du3 = jnp.concatenate(
        [dstp.reshape(NS, _EPW), selfids.reshape(NS, _SPW)],
        axis=1).reshape(NS, _UCH, _UC)

    dst3 = dstp.reshape(NS, _DPW, 128)
    w3 = (jnp.arange(EP, dtype=i32) < E).astype(f32).reshape(NS, _DPW, 128)
    ones = jnp.ones((NP,), f32)

    batchp = jnp.concatenate([batch.astype(i32), jnp.full((NP - N,), NG, i32)])
    batch2d = batchp.reshape(NP // _BM, 1, _BM)

    w1p = jnp.pad(W1.reshape(T, ED, F), ((0, 0), (0, EDP - ED), (0, 0)))
    w1p = jnp.pad(w1p.reshape(KH, F), ((0, 0), (0, FP - F)))
    w1p = w1p.astype(bf16)
    w2p = jnp.pad(W2, ((0, FP - F), (0, FP - F))).astype(bf16)
    b1p2d = jnp.pad(b1, (0, FP - F)).reshape(1, FP)
    b2p2d = jnp.pad(b2, (0, FP - F)).reshape(1, FP)
    l1wp = jnp.pad(lin1_w, ((0, FP - F), (0, 1024 - 1000)))
    l1bp = jnp.pad(lin1_b, (0, 1024 - 1000)).reshape(1, 1024)
    l2wp = jnp.pad(lin2_w, ((0, 1024 - 1000), (0, 128 - 4)))
    l2bp = jnp.pad(lin2_b, (0, 128 - 4)).reshape(1, 128)

    embp = jnp.pad(embed, ((0, 0), (0, EDP - ED)))

    h = _embed_gather(idx3, embp).reshape(NP, KH)
    dis = _degree(dst3, w3, ones)
    dis2d = dis.reshape(NP // _BM, 1, _BM)

    z1 = _mm1(h, w1p, dis2d).reshape(NFB * NP, C)
    acc1 = _propagate(z1, su3, du3)
    z2 = _mm2(acc1, b1p2d, w2p, dis2d).reshape(NFB * NP, C)
    acc2 = _propagate(z2, su3, du3)
    out = _pool_mlp(acc2, b2p2d, dis2d, batch2d, l1wp, l1bp, l2wp, l2bp)
    return out[:, :4]
